# baseline (speedup 1.0000x reference)
import functools

import jax
import jax.numpy as jnp
from jax import lax
from jax.experimental import pallas as pl
from jax.experimental.pallas import tpu as pltpu
from jax.experimental.pallas import tpu_sc as plsc

_INFO = plsc.get_sparse_core_info()
_NC = _INFO.num_cores
_NS = _INFO.num_subcores
_NW = _NC * _NS

_CH = 128
_NBUF = 4
_L = 16


def _gather_call(n_j, D, n_i):
    mesh = plsc.VectorSubcoreMesh(core_axis_name="c", subcore_axis_name="s")
    D2 = 2 * D

    @functools.partial(
        pl.kernel,
        mesh=mesh,
        out_type=jax.ShapeDtypeStruct((n_j, n_i // 2, 2 * D), jnp.float32),
        compiler_params=pltpu.CompilerParams(needs_layout_passes=False),
        scratch_types=[
            pltpu.VMEM((n_j, _CH), jnp.int32),
            pltpu.VMEM((_NBUF, _CH), jnp.int32),
            pltpu.VMEM((_NBUF, _CH, D2), jnp.float32),
            pltpu.VMEM((_NBUF, D, _CH), jnp.float32),
            pltpu.SemaphoreType.DMA((_NBUF,)),
            pltpu.SemaphoreType.DMA((_NBUF,)),
        ],
    )
    def body(idx_hbm, pairs_hbm, out_hbm2, idx_v, pidx_v, pbufs, tbufs,
             gsems, ssems):
        wid = lax.axis_index("s") * _NC + lax.axis_index("c")
        pltpu.sync_copy(idx_hbm.at[:, wid], idx_v)

        def issue(j, b):
            for t in range(_CH // _L):
                sl = pl.ds(t * _L, _L)
                pidx_v[b, sl] = lax.shift_right_logical(idx_v[j, sl], 1)
            pltpu.async_copy(pairs_hbm.at[pidx_v.at[b]], pbufs.at[b],
                             gsems.at[b])

        def wait_gather(b):
            pltpu.make_async_copy(pairs_hbm.at[pl.ds(0, _CH)], pbufs.at[b],
                                  gsems.at[b]).wait()

        def wait_store(b):
            pltpu.make_async_copy(out_hbm2.at[0, pl.ds(0, D), :], tbufs.at[b],
                                  ssems.at[b]).wait()

        ibase = wid * _CH
        lanes = jax.lax.iota(jnp.int32, _L)

        def chunk_body(j, b):
            wait_gather(b)
            for tb in range(_CH // _L):
                tsl = pl.ds(tb * _L, _L)
                ivec = idx_v[j, tsl]
                h64 = lax.shift_left(jnp.bitwise_and(ivec, jnp.int32(1)), 6)
                rows = jnp.int32(tb * _L) + lanes

                @plsc.parallel_loop(0, D, 1, unroll=8)
                def dloop(d, _b=b, _tsl=tsl, _h64=h64, _rows=rows):
                    v = plsc.load_gather(pbufs.at[_b], [_rows, _h64 + d])
                    tbufs[_b, d, _tsl] = v
            pltpu.async_copy(tbufs.at[b],
                             out_hbm2.at[j, pl.ds(wid * D, D), :],
                             ssems.at[b])
            issue((j + _NBUF) % n_j, b)

        for b in range(_NBUF):
            issue(b, b)
        for b in range(_NBUF):
            chunk_body(jnp.int32(b), b)

        def group(g, carry):
            for b in range(_NBUF):
                wait_store(b)
                chunk_body(g * _NBUF + b, b)
            return carry
        lax.fori_loop(1, n_j // _NBUF, group, 0)

        for b in range(_NBUF):
            wait_gather(b)
            wait_store(b)

    return body


def kernel(inlets, weight):
    b0, b1 = inlets.shape
    V, D = weight.shape
    n_i, n_j = b0, b1
    idxT = jnp.swapaxes(inlets, 0, 1).reshape(n_j, _NW, _CH).astype(jnp.int32)
    pairs = weight.reshape(V // 2, 2 * D)
    out3 = _gather_call(n_j, D, n_i)(idxT, pairs)
    out3 = out3.reshape(n_j, D, n_i)
    return jnp.transpose(out3, (2, 0, 1))


# speedup vs baseline: 1.7974x; 1.7974x over previous
import functools

import jax
import jax.numpy as jnp
from jax import lax
from jax.experimental import pallas as pl
from jax.experimental.pallas import tpu as pltpu
from jax.experimental.pallas import tpu_sc as plsc

_INFO = plsc.get_sparse_core_info()
_NC = _INFO.num_cores
_NS = _INFO.num_subcores
_NW = _NC * _NS

_CH = 128
_NBUF = 4
_L = 16


def _gather_call(n_j, D, n_i):
    mesh = plsc.VectorSubcoreMesh(core_axis_name="c", subcore_axis_name="s")
    D2 = 2 * D

    @functools.partial(
        pl.kernel,
        mesh=mesh,
        out_type=jax.ShapeDtypeStruct((n_j, D, n_i), jnp.float32),
        compiler_params=pltpu.CompilerParams(needs_layout_passes=False),
        scratch_types=[
            pltpu.VMEM((n_j, _CH), jnp.int32),
            pltpu.VMEM((_NBUF, _CH), jnp.int32),
            pltpu.VMEM((_NBUF, _CH, D2), jnp.float32),
            pltpu.VMEM((2, D, _CH), jnp.float32),
            pltpu.SemaphoreType.DMA((_NBUF,)),
            pltpu.SemaphoreType.DMA((2,)),
        ],
    )
    def body(idx_hbm, pairs_hbm, out_hbm, idx_v, pidx_v, pbufs, tbufs,
             gsems, ssems):
        wid = lax.axis_index("s") * _NC + lax.axis_index("c")
        pltpu.sync_copy(idx_hbm.at[:, wid], idx_v)

        def issue(j, b):
            for t in range(_CH // _L):
                sl = pl.ds(t * _L, _L)
                pidx_v[b, sl] = lax.shift_right_logical(idx_v[j, sl], 1)
            pltpu.async_copy(pairs_hbm.at[pidx_v.at[b]], pbufs.at[b],
                             gsems.at[b])

        def wait_gather(b):
            pltpu.make_async_copy(pairs_hbm.at[pl.ds(0, _CH)], pbufs.at[b],
                                  gsems.at[b]).wait()

        def wait_store(b2):
            pltpu.make_async_copy(out_hbm.at[0], tbufs.at[b2],
                                  ssems.at[b2]).wait()

        ibase = wid * _CH
        lanes = jax.lax.iota(jnp.int32, _L)

        def chunk_body(j, b, first_round=False):
            b2 = b % 2
            wait_gather(b)
            if not (first_round and b < 2):
                wait_store(b2)
            for tb in range(_CH // _L):
                tsl = pl.ds(tb * _L, _L)
                ivec = idx_v[j, tsl]
                h64 = lax.shift_left(jnp.bitwise_and(ivec, jnp.int32(1)), 6)
                rows = jnp.int32(tb * _L) + lanes

                tvec = jnp.int32(tb * _L) + lanes

                @plsc.parallel_loop(0, D, 1, unroll=8)
                def dloop(d0, _b=b, _b2=b2, _h64=h64, _rows=rows, _tv=tvec):
                    c = jnp.bitwise_and(d0 + lanes, jnp.int32(D - 1))
                    v = plsc.load_gather(pbufs.at[_b], [_rows, _h64 + c])
                    plsc.store_scatter(tbufs.at[_b2], [c, _tv], v)
            pltpu.async_copy(tbufs.at[b2],
                             out_hbm.at[j, :, pl.ds(ibase, _CH)], ssems.at[b2])
            issue((j + _NBUF) % n_j, b)

        for b in range(_NBUF):
            issue(b, b)
        for b in range(_NBUF):
            chunk_body(jnp.int32(b), b, first_round=True)

        def group(g, carry):
            for b in range(_NBUF):
                chunk_body(g * _NBUF + b, b)
            return carry
        lax.fori_loop(1, n_j // _NBUF, group, 0)

        for b in range(_NBUF):
            wait_gather(b)
        for b2 in range(2):
            wait_store(b2)

    return body


def kernel(inlets, weight):
    b0, b1 = inlets.shape
    V, D = weight.shape
    n_i, n_j = b0, b1
    idxT = jnp.swapaxes(inlets, 0, 1).reshape(n_j, _NW, _CH).astype(jnp.int32)
    pairs = weight.reshape(V // 2, 2 * D)
    out3 = _gather_call(n_j, D, n_i)(idxT, pairs)
    return jnp.transpose(out3, (2, 0, 1))


# speedup vs baseline: 2.7466x; 1.5281x over previous
import functools

import jax
import jax.numpy as jnp
from jax import lax
from jax.experimental import pallas as pl
from jax.experimental.pallas import tpu as pltpu
from jax.experimental.pallas import tpu_sc as plsc

_INFO = plsc.get_sparse_core_info()
_NC = _INFO.num_cores
_NS = _INFO.num_subcores
_NW = _NC * _NS

_CH = 128
_NBUF = 4
_L = 16


def _relayout_call(V, D):
    mesh = plsc.VectorSubcoreMesh(core_axis_name="c", subcore_axis_name="s")
    D2 = 2 * D
    EB = 128
    n_eb_full = V // EB
    tail = V - n_eb_full * EB
    G = (n_eb_full + 1 + _NW - 1) // _NW

    @functools.partial(
        pl.kernel,
        mesh=mesh,
        out_type=jax.ShapeDtypeStruct((V // 2, D2), jnp.float32),
        compiler_params=pltpu.CompilerParams(needs_layout_passes=False),
        scratch_types=[
            pltpu.VMEM((2, D, EB), jnp.float32),
            pltpu.VMEM((EB // 2, D2), jnp.float32),
            pltpu.SemaphoreType.DMA((2,)),
        ],
    )
    def body(wt_hbm, tail_hbm, out_hbm, ibufs, obuf, gsems):
        wid = lax.axis_index("s") * _NC + lax.axis_index("c")
        lanes = jax.lax.iota(jnp.int32, _L)

        def eblk(g):
            e = wid + jnp.int32(_NW) * g
            return e, jnp.where(e < n_eb_full, e, 0)

        def issue(g, b):
            _, es = eblk(g)
            pltpu.async_copy(wt_hbm.at[:, pl.ds(es * EB, EB)], ibufs.at[b],
                             gsems.at[b])

        def wait_in(b):
            pltpu.make_async_copy(wt_hbm.at[:, pl.ds(0, EB)], ibufs.at[b],
                                  gsems.at[b]).wait()

        def transpose(b, nk):
            for p0 in range(nk // (2 * _L)):
                pvec = jnp.int32(p0 * _L) + lanes

                @plsc.parallel_loop(0, D2, 1, unroll=8)
                def kloop(k0, _b=b, _pv=pvec):
                    k = jnp.bitwise_and(k0 + lanes, jnp.int32(D2 - 1))
                    dvec = jnp.bitwise_and(k, jnp.int32(D - 1))
                    h = lax.shift_right_logical(k, 6)
                    evec = lax.shift_left(_pv, 1) + h
                    v = plsc.load_gather(ibufs.at[_b], [dvec, evec])
                    plsc.store_scatter(obuf, [_pv, k], v)

        issue(jnp.int32(0), 0)
        issue(jnp.int32(1), 1)

        def step(g, carry):
            b = lax.rem(g, 2)

            def do(b):
                e, _ = eblk(g)
                wait_in(b)
                transpose(b, EB)

                @pl.when(e < n_eb_full)
                def _():
                    pltpu.sync_copy(
                        obuf, out_hbm.at[pl.ds(e * (EB // 2), EB // 2)])
                issue(g + 2, b)

            lax.cond(b == 0, lambda: do(0), lambda: do(1))
            return carry

        lax.fori_loop(0, G, step, 0)
        wait_in(0)
        wait_in(1)

        @pl.when(wid == 0)
        def _():
            pltpu.sync_copy(tail_hbm, obuf.at[pl.ds(0, tail // 2)])
            pltpu.sync_copy(obuf.at[pl.ds(0, tail // 2)],
                            out_hbm.at[pl.ds(n_eb_full * (EB // 2),
                                             tail // 2)])

    return body


def _gather_call(n_j, D, n_i):
    mesh = plsc.VectorSubcoreMesh(core_axis_name="c", subcore_axis_name="s")
    D2 = 2 * D

    @functools.partial(
        pl.kernel,
        mesh=mesh,
        out_type=jax.ShapeDtypeStruct((n_j, D, n_i), jnp.float32),
        compiler_params=pltpu.CompilerParams(needs_layout_passes=False),
        scratch_types=[
            pltpu.VMEM((n_j, _CH), jnp.int32),
            pltpu.VMEM((_NBUF, _CH), jnp.int32),
            pltpu.VMEM((_NBUF, _CH, D2), jnp.float32),
            pltpu.VMEM((2, D, _CH), jnp.float32),
            pltpu.SemaphoreType.DMA((_NBUF,)),
            pltpu.SemaphoreType.DMA((2,)),
        ],
    )
    def body(idx_hbm, pairs_hbm, out_hbm, idx_v, pidx_v, pbufs, tbufs,
             gsems, ssems):
        wid = lax.axis_index("s") * _NC + lax.axis_index("c")
        pltpu.sync_copy(idx_hbm.at[:, wid], idx_v)

        def issue(j, b):
            for t in range(_CH // _L):
                sl = pl.ds(t * _L, _L)
                pidx_v[b, sl] = lax.shift_right_logical(idx_v[j, sl], 1)
            pltpu.async_copy(pairs_hbm.at[pidx_v.at[b]], pbufs.at[b],
                             gsems.at[b])

        def wait_gather(b):
            pltpu.make_async_copy(pairs_hbm.at[pl.ds(0, _CH)], pbufs.at[b],
                                  gsems.at[b]).wait()

        def wait_store(b2):
            pltpu.make_async_copy(out_hbm.at[0], tbufs.at[b2],
                                  ssems.at[b2]).wait()

        ibase = wid * _CH
        lanes = jax.lax.iota(jnp.int32, _L)

        def chunk_body(j, b, first_round=False):
            b2 = b % 2
            wait_gather(b)
            if not (first_round and b < 2):
                wait_store(b2)
            for tb in range(_CH // _L):
                tsl = pl.ds(tb * _L, _L)
                ivec = idx_v[j, tsl]
                h64 = lax.shift_left(jnp.bitwise_and(ivec, jnp.int32(1)), 6)
                rows = jnp.int32(tb * _L) + lanes

                tvec = jnp.int32(tb * _L) + lanes

                @plsc.parallel_loop(0, D, 1, unroll=8)
                def dloop(d0, _b=b, _b2=b2, _h64=h64, _rows=rows, _tv=tvec):
                    c = jnp.bitwise_and(d0 + lanes, jnp.int32(D - 1))
                    v = plsc.load_gather(pbufs.at[_b], [_rows, _h64 + c])
                    plsc.store_scatter(tbufs.at[_b2], [c, _tv], v)
            pltpu.async_copy(tbufs.at[b2],
                             out_hbm.at[j, :, pl.ds(ibase, _CH)], ssems.at[b2])
            issue((j + _NBUF) % n_j, b)

        for b in range(_NBUF):
            issue(b, b)
        for b in range(_NBUF):
            chunk_body(jnp.int32(b), b, first_round=True)

        def group(g, carry):
            for b in range(_NBUF):
                chunk_body(g * _NBUF + b, b)
            return carry
        lax.fori_loop(1, n_j // _NBUF, group, 0)

        for b in range(_NBUF):
            wait_gather(b)
        for b2 in range(2):
            wait_store(b2)

    return body


def kernel(inlets, weight):
    b0, b1 = inlets.shape
    V, D = weight.shape
    n_i, n_j = b0, b1
    idxT = jnp.swapaxes(inlets, 0, 1).reshape(n_j, _NW, _CH).astype(jnp.int32)
    n_tail = V % 128
    tail_pairs = weight[V - n_tail:].reshape(n_tail // 2, 2 * D)
    pairs = _relayout_call(V, D)(jnp.swapaxes(weight, 0, 1), tail_pairs)
    out3 = _gather_call(n_j, D, n_i)(idxT, pairs)
    return jnp.transpose(out3, (2, 0, 1))


# speedup vs baseline: 3.1418x; 1.1439x over previous
import functools

import jax
import jax.numpy as jnp
from jax import lax
from jax.experimental import pallas as pl
from jax.experimental.pallas import tpu as pltpu
from jax.experimental.pallas import tpu_sc as plsc

_INFO = plsc.get_sparse_core_info()
_NC = _INFO.num_cores
_NS = _INFO.num_subcores
_NW = _NC * _NS

_CH = 128
_NBUF = 4
_L = 16


def _relayout_call(V, D):
    mesh = plsc.VectorSubcoreMesh(core_axis_name="c", subcore_axis_name="s")
    D2 = 2 * D
    EB = 128
    n_eb_full = V // EB
    tail = V - n_eb_full * EB
    G = (n_eb_full + 1 + _NW - 1) // _NW

    @functools.partial(
        pl.kernel,
        mesh=mesh,
        out_type=jax.ShapeDtypeStruct((V // 2, D2), jnp.float32),
        compiler_params=pltpu.CompilerParams(needs_layout_passes=False),
        scratch_types=[
            pltpu.VMEM((3, D, EB), jnp.float32),
            pltpu.VMEM((EB // 2, D2), jnp.float32),
            pltpu.SemaphoreType.DMA((3,)),
        ],
    )
    def body(wt_hbm, tail_hbm, out_hbm, ibufs, obuf, gsems):
        wid = lax.axis_index("s") * _NC + lax.axis_index("c")
        lanes = jax.lax.iota(jnp.int32, _L)

        def eblk(g):
            e = wid + jnp.int32(_NW) * g
            return e, jnp.where(e < n_eb_full, e, 0)

        def issue(g, b):
            _, es = eblk(g)
            pltpu.async_copy(wt_hbm.at[:, pl.ds(es * EB, EB)], ibufs.at[b],
                             gsems.at[b])

        def wait_in(b):
            pltpu.make_async_copy(wt_hbm.at[:, pl.ds(0, EB)], ibufs.at[b],
                                  gsems.at[b]).wait()

        def transpose(b, nk):
            for p0 in range(nk // (2 * _L)):
                pvec = jnp.int32(p0 * _L) + lanes
                for h in range(2):
                    evec = lax.shift_left(pvec, 1) + jnp.int32(h)
                    hD = jnp.int32(h * D)

                    @plsc.parallel_loop(0, D, 1, unroll=8)
                    def dloop(d0, _b=b, _pv=pvec, _ev=evec, _hD=hD):
                        dvec = jnp.bitwise_and(d0 + lanes, jnp.int32(D - 1))
                        v = plsc.load_gather(ibufs.at[_b], [dvec, _ev])
                        plsc.store_scatter(obuf, [_pv, dvec + _hD], v)

        for b in range(3):
            issue(jnp.int32(b), b)

        def group(gg, carry):
            for b in range(3):
                g = gg * 3 + b
                e, _ = eblk(g)
                wait_in(b)
                transpose(b, EB)

                @pl.when(e < n_eb_full)
                def _():
                    pltpu.sync_copy(
                        obuf, out_hbm.at[pl.ds(e * (EB // 2), EB // 2)])
                issue(g + 3, b)
            return carry

        lax.fori_loop(0, (G + 2) // 3, group, 0)
        for b in range(3):
            wait_in(b)

        @pl.when(wid == 0)
        def _():
            pltpu.sync_copy(tail_hbm, obuf.at[pl.ds(0, tail // 2)])
            pltpu.sync_copy(obuf.at[pl.ds(0, tail // 2)],
                            out_hbm.at[pl.ds(n_eb_full * (EB // 2),
                                             tail // 2)])

    return body


def _gather_call(n_j, D, n_i):
    mesh = plsc.VectorSubcoreMesh(core_axis_name="c", subcore_axis_name="s")
    D2 = 2 * D

    @functools.partial(
        pl.kernel,
        mesh=mesh,
        out_type=jax.ShapeDtypeStruct((n_j, D, n_i), jnp.float32),
        compiler_params=pltpu.CompilerParams(needs_layout_passes=False),
        scratch_types=[
            pltpu.VMEM((n_j, _CH), jnp.int32),
            pltpu.VMEM((_NBUF, _CH), jnp.int32),
            pltpu.VMEM((_NBUF, _CH, D2), jnp.float32),
            pltpu.VMEM((2, D, _CH), jnp.float32),
            pltpu.SemaphoreType.DMA((_NBUF,)),
            pltpu.SemaphoreType.DMA((2,)),
        ],
    )
    def body(idx_hbm, pairs_hbm, out_hbm, idx_v, pidx_v, pbufs, tbufs,
             gsems, ssems):
        wid = lax.axis_index("s") * _NC + lax.axis_index("c")
        pltpu.sync_copy(idx_hbm.at[:, wid], idx_v)

        def issue(j, b):
            for t in range(_CH // _L):
                sl = pl.ds(t * _L, _L)
                pidx_v[b, sl] = lax.shift_right_logical(idx_v[j, sl], 1)
            pltpu.async_copy(pairs_hbm.at[pidx_v.at[b]], pbufs.at[b],
                             gsems.at[b])

        def wait_gather(b):
            pltpu.make_async_copy(pairs_hbm.at[pl.ds(0, _CH)], pbufs.at[b],
                                  gsems.at[b]).wait()

        def wait_store(b2):
            pltpu.make_async_copy(out_hbm.at[0], tbufs.at[b2],
                                  ssems.at[b2]).wait()

        ibase = wid * _CH
        lanes = jax.lax.iota(jnp.int32, _L)

        def chunk_body(j, b, first_round=False):
            b2 = b % 2
            wait_gather(b)
            if not (first_round and b < 2):
                wait_store(b2)
            for tb in range(_CH // _L):
                tsl = pl.ds(tb * _L, _L)
                ivec = idx_v[j, tsl]
                h64 = lax.shift_left(jnp.bitwise_and(ivec, jnp.int32(1)), 6)
                rows = jnp.int32(tb * _L) + lanes

                tvec = jnp.int32(tb * _L) + lanes

                @plsc.parallel_loop(0, D, 1, unroll=8)
                def dloop(d0, _b=b, _b2=b2, _h64=h64, _rows=rows, _tv=tvec):
                    c = jnp.bitwise_and(d0 + lanes, jnp.int32(D - 1))
                    v = plsc.load_gather(pbufs.at[_b], [_rows, _h64 + c])
                    plsc.store_scatter(tbufs.at[_b2], [c, _tv], v)
            pltpu.async_copy(tbufs.at[b2],
                             out_hbm.at[j, :, pl.ds(ibase, _CH)], ssems.at[b2])
            issue((j + _NBUF) % n_j, b)

        for b in range(_NBUF):
            issue(b, b)
        for b in range(_NBUF):
            chunk_body(jnp.int32(b), b, first_round=True)

        def group(g, carry):
            for b in range(_NBUF):
                chunk_body(g * _NBUF + b, b)
            return carry
        lax.fori_loop(1, n_j // _NBUF, group, 0)

        for b in range(_NBUF):
            wait_gather(b)
        for b2 in range(2):
            wait_store(b2)

    return body


def kernel(inlets, weight):
    b0, b1 = inlets.shape
    V, D = weight.shape
    n_i, n_j = b0, b1
    idxT = jnp.swapaxes(inlets, 0, 1).reshape(n_j, _NW, _CH).astype(jnp.int32)
    n_tail = V % 128
    tail_pairs = weight[V - n_tail:].reshape(n_tail // 2, 2 * D)
    pairs = _relayout_call(V, D)(jnp.swapaxes(weight, 0, 1), tail_pairs)
    out3 = _gather_call(n_j, D, n_i)(idxT, pairs)
    return jnp.transpose(out3, (2, 0, 1))
